# baseline (speedup 1.0000x reference)
import functools

import jax
import jax.numpy as jnp
from jax import lax
from jax.experimental import pallas as pl
from jax.experimental.pallas import tpu as pltpu
from jax.experimental.pallas import tpu_sc as plsc

_NC, _NS = 2, 16
_NW = _NC * _NS
_D = 64
_SEG = 128
_T8 = 25
_B32 = 32
_UPW = _T8 * _B32 // _NW


def _gather_body(ids_hbm, table_hbm, out_hbm, dummy_hbm,
                 idx_v, g_v, t_v, isem, gsem0, gsem1, osem0, osem1):
    gsem = (gsem0, gsem1)
    osem = (osem0, osem1)
    wid = lax.axis_index("s") * _NC + lax.axis_index("c")

    cps = []
    for k in range(_UPW):
        u = wid * _UPW + k
        cps.append(pltpu.async_copy(
            ids_hbm.at[u // _B32, u % _B32], idx_v.at[k], isem))
    for cp in cps:
        cp.wait()

    pltpu.async_copy(t_v.at[0], dummy_hbm.at[wid], osem[0])
    pltpu.async_copy(t_v.at[1], dummy_hbm.at[wid], osem[1])

    pltpu.async_copy(table_hbm.at[idx_v.at[0, 0]], g_v.at[0], gsem[0])

    def wait_gather(gb):
        pltpu.make_async_copy(
            table_hbm.at[idx_v.at[0, 0]], g_v.at[gb], gsem[gb]).wait()

    def wait_wb(ob, k, b32, it):
        pltpu.make_async_copy(
            t_v.at[ob], out_hbm.at[0, :, b32], osem[ob]).wait()

    def extract(gb, ob):
        jvs = [lax.iota(jnp.int32, 16) + jg * 16 for jg in range(8)]

        def dbody(d8, carry):
            for i in range(8):
                dv = jnp.full((16,), d8 * 8 + i, jnp.int32)
                for jg in range(8):
                    vals = plsc.load_gather(g_v.at[gb], [jvs[jg], dv])
                    t_v[ob, d8, i, pl.ds(jg * 16, 16)] = vals
            return carry

        lax.fori_loop(0, 8, dbody, 0)

    def unit_body(k, carry):
        u = wid * _UPW + k
        t8 = u // _B32
        b32 = u % _B32
        for it in range(8):
            gb = it % 2
            wait_gather(gb)
            if it < 7:
                pltpu.async_copy(
                    table_hbm.at[idx_v.at[k, it + 1]], g_v.at[1 - gb],
                    gsem[1 - gb])
            else:
                @pl.when(k < _UPW - 1)
                def _():
                    u2 = u + 1
                    pltpu.async_copy(
                        table_hbm.at[idx_v.at[k + 1, 0]], g_v.at[1 - gb],
                        gsem[1 - gb])
            wait_wb(gb, k, b32, it)
            extract(gb, gb)
            pltpu.async_copy(
                t_v.at[gb], out_hbm.at[t8 * 8 + it, :, b32], osem[gb])
        return carry

    lax.fori_loop(0, _UPW, unit_body, 0)

    wait_wb(0, 0, 0, 0)
    wait_wb(1, 0, 0, 0)


def _sc_gather(ids_view, t2):
    kfn = functools.partial(
        pl.kernel,
        mesh=plsc.VectorSubcoreMesh(core_axis_name="c", subcore_axis_name="s"),
        out_type=(
            jax.ShapeDtypeStruct((200, 8, _B32, 8, _SEG), jnp.float32),
            jax.ShapeDtypeStruct((_NW, 8, 8, _SEG), jnp.float32),
        ),
        scratch_types=[
            pltpu.VMEM((_UPW, 8, _SEG), jnp.int32),
            pltpu.VMEM((2, _SEG, _D), jnp.float32),
            pltpu.VMEM((2, 8, 8, _SEG), jnp.float32),
        ] + [pltpu.SemaphoreType.DMA] * 5,
        compiler_params=pltpu.CompilerParams(
            use_tc_tiling_on_sc=False, needs_layout_passes=False),
    )(_gather_body)
    return kfn(ids_view, t2)


def _mask_body(ids_ref, mask_ref):
    mask_ref[...] = ids_ref[...] != 0


def _tc_mask(ids):
    return pl.pallas_call(
        _mask_body,
        out_shape=jax.ShapeDtypeStruct(ids.shape, jnp.bool_),
    )(ids)


def kernel(ids, table):
    ids_view = ids.reshape(_B32, _SEG, _T8, 8).transpose(2, 0, 3, 1)
    out5d, _ = _sc_gather(ids_view, table)
    tokens = out5d.transpose(2, 4, 0, 1, 3).reshape(4096, 200, _D)
    mask = _tc_mask(ids)
    return tokens, mask


# speedup vs baseline: 1.4678x; 1.4678x over previous
import functools

import jax
import jax.numpy as jnp
from jax import lax
from jax.experimental import pallas as pl
from jax.experimental.pallas import tpu as pltpu
from jax.experimental.pallas import tpu_sc as plsc

_NC, _NS = 2, 16
_NW = _NC * _NS
_D = 64
_SEG = 128
_T8 = 25
_B32 = 32
_UPW = _T8 * _B32 // _NW


def _gather_body(ids_hbm, table_hbm, out_hbm, dummy_hbm,
                 idx_v, g_v, t_v, isem, gsem0, gsem1, osem0, osem1):
    gsem = (gsem0, gsem1)
    osem = (osem0, osem1)
    wid = lax.axis_index("s") * _NC + lax.axis_index("c")

    cps = []
    for k in range(_UPW):
        u = wid * _UPW + k
        cps.append(pltpu.async_copy(
            ids_hbm.at[u // _B32, u % _B32], idx_v.at[k], isem))
    for cp in cps:
        cp.wait()

    pltpu.async_copy(t_v.at[0], dummy_hbm.at[wid], osem[0])
    pltpu.async_copy(t_v.at[1], dummy_hbm.at[wid], osem[1])

    pltpu.async_copy(table_hbm.at[idx_v.at[0, 0]], g_v.at[0], gsem[0])

    def wait_gather(gb):
        pltpu.make_async_copy(
            table_hbm.at[idx_v.at[0, 0]], g_v.at[gb], gsem[gb]).wait()

    def wait_wb(ob, k, b32, it):
        pltpu.make_async_copy(
            t_v.at[ob], out_hbm.at[0, :, b32], osem[ob]).wait()

    def extract(gb, ob):
        jvs = [lax.iota(jnp.int32, 16) + jg * 16 for jg in range(8)]

        @plsc.parallel_loop(0, _D, step=1, unroll=4)
        def _(d):
            dv = jnp.full((16,), d, jnp.int32)
            for jg in range(8):
                vals = plsc.load_gather(g_v.at[gb], [jvs[jg], dv])
                t_v[ob, d // 8, d % 8, pl.ds(jg * 16, 16)] = vals

    def unit_body(k, carry):
        u = wid * _UPW + k
        t8 = u // _B32
        b32 = u % _B32
        for it in range(8):
            gb = it % 2
            wait_gather(gb)
            if it < 7:
                pltpu.async_copy(
                    table_hbm.at[idx_v.at[k, it + 1]], g_v.at[1 - gb],
                    gsem[1 - gb])
            else:
                @pl.when(k < _UPW - 1)
                def _():
                    u2 = u + 1
                    pltpu.async_copy(
                        table_hbm.at[idx_v.at[k + 1, 0]], g_v.at[1 - gb],
                        gsem[1 - gb])
            wait_wb(gb, k, b32, it)
            extract(gb, gb)
            pltpu.async_copy(
                t_v.at[gb], out_hbm.at[t8 * 8 + it, :, b32], osem[gb])
        return carry

    lax.fori_loop(0, _UPW, unit_body, 0)

    wait_wb(0, 0, 0, 0)
    wait_wb(1, 0, 0, 0)


def _sc_gather(ids_view, t2):
    kfn = functools.partial(
        pl.kernel,
        mesh=plsc.VectorSubcoreMesh(core_axis_name="c", subcore_axis_name="s"),
        out_type=(
            jax.ShapeDtypeStruct((200, 8, _B32, 8, _SEG), jnp.float32),
            jax.ShapeDtypeStruct((_NW, 8, 8, _SEG), jnp.float32),
        ),
        scratch_types=[
            pltpu.VMEM((_UPW, 8, _SEG), jnp.int32),
            pltpu.VMEM((2, _SEG, _D), jnp.float32),
            pltpu.VMEM((2, 8, 8, _SEG), jnp.float32),
        ] + [pltpu.SemaphoreType.DMA] * 5,
        compiler_params=pltpu.CompilerParams(
            use_tc_tiling_on_sc=False, needs_layout_passes=False),
    )(_gather_body)
    return kfn(ids_view, t2)


def _mask_body(ids_ref, mask_ref):
    mask_ref[...] = ids_ref[...] != 0


def _tc_mask(ids):
    return pl.pallas_call(
        _mask_body,
        out_shape=jax.ShapeDtypeStruct(ids.shape, jnp.bool_),
    )(ids)


def kernel(ids, table):
    ids_view = ids.reshape(_B32, _SEG, _T8, 8).transpose(2, 0, 3, 1)
    out5d, _ = _sc_gather(ids_view, table)
    tokens = out5d.transpose(2, 4, 0, 1, 3).reshape(4096, 200, _D)
    mask = _tc_mask(ids)
    return tokens, mask


# speedup vs baseline: 1.4865x; 1.0127x over previous
import functools

import jax
import jax.numpy as jnp
from jax import lax
from jax.experimental import pallas as pl
from jax.experimental.pallas import tpu as pltpu
from jax.experimental.pallas import tpu_sc as plsc

_NC, _NS = 2, 16
_NW = _NC * _NS
_D = 64
_SEG = 128
_RING = 8


def _gather_body(n_streams, ids_hbm, table_hbm, out_hbm, idx_v, rows_v, *sems):
    gsem = sems[:_RING]
    osem = sems[_RING:]
    wid = lax.axis_index("s") * _NC + lax.axis_index("c")
    n_rounds = n_streams // _RING

    def fire_gather(b, s):
        return pltpu.async_copy(table_hbm.at[idx_v.at[s]], rows_v.at[b], gsem[b])

    def wait_gather(b, s):
        pltpu.make_async_copy(table_hbm.at[idx_v.at[s]], rows_v.at[b], gsem[b]).wait()

    def fire_wb(b, s):
        return pltpu.async_copy(rows_v.at[b], out_hbm.at[wid, s], osem[b])

    def wait_wb(b, s):
        pltpu.make_async_copy(rows_v.at[b], out_hbm.at[wid, s], osem[b]).wait()

    pltpu.sync_copy(ids_hbm.at[wid], idx_v)

    for b in range(_RING):
        fire_gather(b, b)

    def round_body(r, carry):
        for b in range(_RING):
            s = r * _RING + b
            wait_gather(b, s)
            fire_wb(b, s)
        for b in range(_RING):
            s = r * _RING + b
            wait_wb(b, s)
            fire_gather(b, s + _RING)
        return carry

    lax.fori_loop(0, n_rounds - 1, round_body, 0)

    r = n_rounds - 1
    for b in range(_RING):
        s = r * _RING + b
        wait_gather(b, s)
        fire_wb(b, s)
    for b in range(_RING):
        wait_wb(b, r * _RING + b)


def _sc_gather(ids_r, table, n_streams):
    kfn = functools.partial(
        pl.kernel,
        mesh=plsc.VectorSubcoreMesh(core_axis_name="c", subcore_axis_name="s"),
        out_type=jax.ShapeDtypeStruct((_NW, n_streams, _SEG, _D), jnp.float32),
        scratch_types=[
            pltpu.VMEM((n_streams, _SEG), jnp.int32),
            pltpu.VMEM((_RING, _SEG, _D), jnp.float32),
        ] + [pltpu.SemaphoreType.DMA] * (2 * _RING),
        compiler_params=pltpu.CompilerParams(use_tc_tiling_on_sc=False),
    )(functools.partial(_gather_body, n_streams))
    return kfn(ids_r, table)


def _mask_body(ids_ref, mask_ref):
    mask_ref[...] = ids_ref[...] != 0


def _tc_mask(ids):
    return pl.pallas_call(
        _mask_body,
        out_shape=jax.ShapeDtypeStruct(ids.shape, jnp.bool_),
    )(ids)


def kernel(ids, table):
    b, t = ids.shape
    total = b * t
    n_streams = total // (_NW * _SEG)
    assert n_streams * _NW * _SEG == total and n_streams % _RING == 0
    ids_r = ids.reshape(_NW, n_streams, _SEG)
    tokens = _sc_gather(ids_r, table, n_streams).reshape(b, t, _D)
    mask = _tc_mask(ids)
    return tokens, mask
